# baseline (speedup 1.0000x reference)
import functools

import jax
import jax.numpy as jnp
from jax import lax
from jax.experimental import pallas as pl
from jax.experimental.pallas import tpu as pltpu
from jax.experimental.pallas import tpu_sc as plsc

PI = 3.141592653589793
EPS = 1e-7

B, A, NC = 16, 21504, 80
AS = 12288
ABLK1 = 3072
GRID1 = (A - AS) // ABLK1
OFF1 = AS // ABLK1
AB2 = 3072
GRID2 = AS // AB2
NW = 32
WPER = AS // NW
CHK = 128
NCHUNK = WPER // CHK
RG = 160
NGRP = (B * NC) // RG


def _acos(x):
    ax = jnp.abs(x)
    p = (1.5707963050 + ax * (-0.2145988016 + ax * (0.0889789874 + ax * (
        -0.0501743046 + ax * (0.0308918810 + ax * (-0.0170881256 + ax * (
            0.0066700901 + ax * -0.0012624911)))))))
    r = jnp.sqrt(jnp.maximum(1.0 - ax, 0.0)) * p
    return jnp.where(x >= 0.0, r, PI - r)


def _circle_losses(x1, y1, r1, x2, y2, r2):
    d2 = (x1 - x2) ** 2 + (y1 - y2) ** 2
    d = jnp.sqrt(jnp.maximum(d2, EPS))
    rsum = r1 + r2
    rdiff = jnp.abs(r1 - r2)
    rmin = jnp.minimum(r1, r2)
    no_overlap = d >= rsum
    contained = d <= rdiff
    a1 = jnp.clip((d2 + r1 ** 2 - r2 ** 2) / (2.0 * d * jnp.maximum(r1, EPS)),
                  -1.0 + 1e-6, 1.0 - 1e-6)
    a2 = jnp.clip((d2 + r2 ** 2 - r1 ** 2) / (2.0 * d * jnp.maximum(r2, EPS)),
                  -1.0 + 1e-6, 1.0 - 1e-6)
    tri = jnp.maximum((-d + rsum) * (d + r1 - r2) * (d - r1 + r2) * (d + rsum),
                      EPS)
    lens = (r1 ** 2 * _acos(a1) + r2 ** 2 * _acos(a2)
            - 0.5 * jnp.sqrt(tri))
    inter = jnp.where(no_overlap, 0.0, jnp.where(contained, PI * rmin ** 2, lens))
    union = PI * (r1 ** 2 + r2 ** 2) - inter
    iou = inter / (union + EPS)
    dist = jnp.clip(1.0 - d / (rsum + EPS), 0.0, 1.0)
    return iou, dist


def _sc_w_body(st2_hbm, w_hbm, buf, wbuf, sem0, sem1, semo):
    wid = lax.axis_index("s") * 2 + lax.axis_index("c")
    base = wid * WPER
    sems = (sem0, sem1)
    ng = NCHUNK * NGRP

    def mk(gi):
        k, g = divmod(gi, NGRP)
        src = st2_hbm.at[pl.ds(RG * g, RG), pl.ds(base + CHK * k, CHK)]
        return pltpu.make_async_copy(src, buf.at[gi % 2], sems[gi % 2])

    mk(0).start()
    for gi in range(ng):
        if gi + 1 < ng:
            mk(gi + 1).start()
        mk(gi).wait()
        k, g = divmod(gi, NGRP)
        p = gi % 2
        for half in range(2):
            b = 2 * g + half

            def row_body(r, accs):
                return tuple(
                    accs[v] + buf[p, 80 * half + r, pl.ds(16 * v, 16)]
                    for v in range(CHK // 16))
            accs = lax.fori_loop(
                0, NC, row_body,
                tuple(jnp.zeros((16,), jnp.float32)
                      for _ in range(CHK // 16)),
                unroll=8)
            for v in range(CHK // 16):
                wbuf[b, pl.ds(16 * v, 16)] = accs[v]
        if g == NGRP - 1:
            cpo = pltpu.make_async_copy(
                wbuf, w_hbm.at[:, pl.ds(base + CHK * k, CHK)], semo)
            cpo.start()
            cpo.wait()


@jax.jit
def _sc_w(st2):
    f = pl.kernel(
        _sc_w_body,
        out_type=jax.ShapeDtypeStruct((B, AS), jnp.float32),
        mesh=plsc.VectorSubcoreMesh(core_axis_name="c", subcore_axis_name="s"),
        compiler_params=pltpu.CompilerParams(use_tc_tiling_on_sc=True),
        scratch_types=[
            pltpu.VMEM((2, RG, CHK), jnp.float32),
            pltpu.VMEM((B, CHK), jnp.float32),
            pltpu.SemaphoreType.DMA,
            pltpu.SemaphoreType.DMA,
            pltpu.SemaphoreType.DMA,
        ],
    )
    return f(st2)


def _tc1_body(s_ref, p_ref, t_ref, m_ref, iou_out, dist_out):
    i = pl.program_id(0)

    @pl.when(i == 0)
    def _init():
        iou_out[0, 0] = 0.0
        dist_out[0, 0] = 0.0

    w = jnp.sum(s_ref[...], axis=1)
    m = m_ref[...]
    iou, dist = _circle_losses(
        p_ref[:, 0, :], p_ref[:, 1, :], p_ref[:, 2, :],
        t_ref[:, 0, :], t_ref[:, 1, :], t_ref[:, 2, :])
    wm = w * m
    iou_out[0, 0] += jnp.sum((1.0 - iou) * wm)
    dist_out[0, 0] += jnp.sum((1.0 - dist) * wm)


@jax.jit
def _tc1(st, pt, tt, mt):
    return pl.pallas_call(
        _tc1_body,
        grid=(GRID1,),
        in_specs=[
            pl.BlockSpec((B, NC, ABLK1), lambda i: (0, 0, i + OFF1)),
            pl.BlockSpec((B, 3, ABLK1), lambda i: (0, 0, i + OFF1)),
            pl.BlockSpec((B, 3, ABLK1), lambda i: (0, 0, i + OFF1)),
            pl.BlockSpec((B, ABLK1), lambda i: (0, i + OFF1)),
        ],
        out_specs=[
            pl.BlockSpec(memory_space=pltpu.SMEM),
            pl.BlockSpec(memory_space=pltpu.SMEM),
        ],
        out_shape=[
            jax.ShapeDtypeStruct((1, 1), jnp.float32),
            jax.ShapeDtypeStruct((1, 1), jnp.float32),
        ],
    )(st, pt, tt, mt)


def _tc2_body(w_ref, p_ref, t_ref, m_ref, iou_out, dist_out):
    i = pl.program_id(0)

    @pl.when(i == 0)
    def _init():
        iou_out[0, 0] = 0.0
        dist_out[0, 0] = 0.0

    w = w_ref[...]
    m = m_ref[...]
    iou, dist = _circle_losses(
        p_ref[:, 0, :], p_ref[:, 1, :], p_ref[:, 2, :],
        t_ref[:, 0, :], t_ref[:, 1, :], t_ref[:, 2, :])
    wm = w * m
    iou_out[0, 0] += jnp.sum((1.0 - iou) * wm)
    dist_out[0, 0] += jnp.sum((1.0 - dist) * wm)


@jax.jit
def _tc2(w0, pt, tt, mt):
    return pl.pallas_call(
        _tc2_body,
        grid=(GRID2,),
        in_specs=[
            pl.BlockSpec((B, AB2), lambda i: (0, i)),
            pl.BlockSpec((B, 3, AB2), lambda i: (0, 0, i)),
            pl.BlockSpec((B, 3, AB2), lambda i: (0, 0, i)),
            pl.BlockSpec((B, AB2), lambda i: (0, i)),
        ],
        out_specs=[
            pl.BlockSpec(memory_space=pltpu.SMEM),
            pl.BlockSpec(memory_space=pltpu.SMEM),
        ],
        out_shape=[
            jax.ShapeDtypeStruct((1, 1), jnp.float32),
            jax.ShapeDtypeStruct((1, 1), jnp.float32),
        ],
    )(w0, pt, tt, mt)


def kernel(pred_dist, pred_bboxes, anchor_points, target_bboxes,
           target_scores, target_scores_sum, fg_mask):
    st = jnp.transpose(target_scores, (0, 2, 1))
    st2 = st.reshape(B * NC, A)
    pt = jnp.transpose(pred_bboxes, (0, 2, 1))
    tt = jnp.transpose(target_bboxes, (0, 2, 1))
    mt = fg_mask.astype(jnp.float32)

    w0 = _sc_w(st2)
    si1, sd1 = _tc1(st, pt, tt, mt)
    si2, sd2 = _tc2(w0, pt, tt, mt)

    inv = 1.0 / target_scores_sum
    return ((si1[0, 0] + si2[0, 0]) * inv, (sd1[0, 0] + sd2[0, 0]) * inv)


# speedup vs baseline: 1.4633x; 1.4633x over previous
import jax
import jax.numpy as jnp
from jax import lax
from jax.experimental import pallas as pl
from jax.experimental.pallas import tpu as pltpu

PI = 3.141592653589793
EPS = 1e-7

B, A, NC = 16, 21504, 80
ABLK = 2688
GRID = A // ABLK
NCH = NC // 2


def _acos(x):
    ax = jnp.abs(x)
    p = (1.5707963050 + ax * (-0.2145988016 + ax * (0.0889789874 + ax * (
        -0.0501743046 + ax * (0.0308918810 + ax * (-0.0170881256 + ax * (
            0.0066700901 + ax * -0.0012624911)))))))
    r = jnp.sqrt(jnp.maximum(1.0 - ax, 0.0)) * p
    return jnp.where(x >= 0.0, r, PI - r)


def _circle_losses(x1, y1, r1, x2, y2, r2):
    d2 = (x1 - x2) ** 2 + (y1 - y2) ** 2
    d = jnp.sqrt(jnp.maximum(d2, EPS))
    rsum = r1 + r2
    rdiff = jnp.abs(r1 - r2)
    rmin = jnp.minimum(r1, r2)
    no_overlap = d >= rsum
    contained = d <= rdiff
    a1 = jnp.clip((d2 + r1 ** 2 - r2 ** 2) / (2.0 * d * jnp.maximum(r1, EPS)),
                  -1.0 + 1e-6, 1.0 - 1e-6)
    a2 = jnp.clip((d2 + r2 ** 2 - r1 ** 2) / (2.0 * d * jnp.maximum(r2, EPS)),
                  -1.0 + 1e-6, 1.0 - 1e-6)
    tri = jnp.maximum((-d + rsum) * (d + r1 - r2) * (d - r1 + r2) * (d + rsum),
                      EPS)
    lens = (r1 ** 2 * _acos(a1) + r2 ** 2 * _acos(a2)
            - 0.5 * jnp.sqrt(tri))
    inter = jnp.where(no_overlap, 0.0, jnp.where(contained, PI * rmin ** 2, lens))
    union = PI * (r1 ** 2 + r2 ** 2) - inter
    iou = inter / (union + EPS)
    dist = jnp.clip(1.0 - d / (rsum + EPS), 0.0, 1.0)
    return iou, dist


def _loss_body(s1_ref, s2_ref, p_ref, t_ref, m_ref, iou_out, dist_out):
    i = pl.program_id(0)

    @pl.when(i == 0)
    def _init():
        iou_out[0, 0] = 0.0
        dist_out[0, 0] = 0.0

    w = jnp.sum(s1_ref[...], axis=1) + jnp.sum(s2_ref[...], axis=1)
    m = m_ref[...]
    iou, dist = _circle_losses(
        p_ref[:, 0, :], p_ref[:, 1, :], p_ref[:, 2, :],
        t_ref[:, 0, :], t_ref[:, 1, :], t_ref[:, 2, :])
    wm = w * m
    iou_out[0, 0] += jnp.sum((1.0 - iou) * wm)
    dist_out[0, 0] += jnp.sum((1.0 - dist) * wm)


@jax.jit
def _loss_sums(st, pt, tt, mt):
    return pl.pallas_call(
        _loss_body,
        grid=(GRID,),
        in_specs=[
            pl.BlockSpec((B, NCH, ABLK), lambda i: (0, 0, i)),
            pl.BlockSpec((B, NCH, ABLK), lambda i: (0, 1, i)),
            pl.BlockSpec((B, 3, ABLK), lambda i: (0, 0, i)),
            pl.BlockSpec((B, 3, ABLK), lambda i: (0, 0, i)),
            pl.BlockSpec((B, ABLK), lambda i: (0, i)),
        ],
        out_specs=[
            pl.BlockSpec(memory_space=pltpu.SMEM),
            pl.BlockSpec(memory_space=pltpu.SMEM),
        ],
        out_shape=[
            jax.ShapeDtypeStruct((1, 1), jnp.float32),
            jax.ShapeDtypeStruct((1, 1), jnp.float32),
        ],
    )(st, st, pt, tt, mt)


def kernel(pred_dist, pred_bboxes, anchor_points, target_bboxes,
           target_scores, target_scores_sum, fg_mask):
    st = jnp.transpose(target_scores, (0, 2, 1))
    pt = jnp.transpose(pred_bboxes, (0, 2, 1))
    tt = jnp.transpose(target_bboxes, (0, 2, 1))
    mt = fg_mask.astype(jnp.float32)
    si, sd = _loss_sums(st, pt, tt, mt)
    inv = 1.0 / target_scores_sum
    return (si[0, 0] * inv, sd[0, 0] * inv)
